# Initial kernel scaffold; baseline (speedup 1.0000x reference)
import jax
import jax.numpy as jnp
from jax.experimental import pallas as pl


def kernel(x):
    raise NotImplementedError("write your pallas kernel here")


# speedup vs baseline: 1.9426x; 1.9426x over previous
import functools

import jax
import jax.numpy as jnp
from jax import lax
from jax.experimental import pallas as pl
from jax.experimental.pallas import tpu as pltpu
from jax.experimental.pallas import tpu_sc as plsc

ROWS = 64
N = 32768
LANES = 16
CHUNK = N // LANES
NC, NS = 2, 16
NWORKERS = NC * NS
ROWS_PER_W = ROWS // NWORKERS
RADIX = 256
HIST = RADIX * LANES


def _sort_body(x_hbm, out_hbm, buf_a, buf_b, hist):
    lane = lax.iota(jnp.int32, LANES)
    base_idx = lane * CHUNK
    ones = jnp.ones((LANES,), jnp.int32)
    zeros = jnp.zeros((LANES,), jnp.int32)
    msb = jnp.full((LANES,), -2147483648, jnp.int32)
    c31 = jnp.full((LANES,), 31, jnp.int32)
    m255 = jnp.full((LANES,), 255, jnp.int32)
    c16 = jnp.full((LANES,), LANES, jnp.int32)

    def digit_of(keys_f32, shift):
        k = plsc.bitcast(keys_f32, jnp.int32)
        m = lax.shift_right_arithmetic(k, c31)
        u = lax.bitwise_xor(k, lax.bitwise_or(m, msb))
        if shift:
            u = lax.shift_right_logical(u, jnp.full((LANES,), shift, jnp.int32))
        return lax.bitwise_and(u, m255)

    wid = lax.axis_index("s") * NC + lax.axis_index("c")

    for r in range(ROWS_PER_W):
        row = wid * ROWS_PER_W + r
        pltpu.sync_copy(x_hbm.at[row], buf_a)

        for p, (src, dst) in enumerate(
            [(buf_a, buf_b), (buf_b, buf_a), (buf_a, buf_b), (buf_b, buf_a)]
        ):
            shift = 8 * p

            def zero_body(i, carry):
                hist[pl.ds(i * LANES, LANES)] = zeros
                return carry

            lax.fori_loop(0, HIST // LANES, zero_body, 0, unroll=4)

            def hist_body(j, carry):
                keys = plsc.load_gather(src, [base_idx + j])
                d = digit_of(keys, shift)
                addr = d * c16 + lane
                plsc.addupdate_scatter(hist, [addr], ones)
                return carry

            lax.fori_loop(0, CHUNK, hist_body, 0, unroll=4)

            def scan_body(i, c):
                h = hist[pl.ds(i * LANES, LANES)]
                incl = plsc.cumsum(h)
                hist[pl.ds(i * LANES, LANES)] = incl - h + c
                return c + jnp.sum(h)

            lax.fori_loop(0, HIST // LANES, scan_body, jnp.int32(0))

            def perm_body(j, carry):
                keys = plsc.load_gather(src, [base_idx + j])
                d = digit_of(keys, shift)
                addr = d * c16 + lane
                off = plsc.load_gather(hist, [addr])
                plsc.store_scatter(dst, [off], keys)
                plsc.addupdate_scatter(hist, [addr], ones)
                return carry

            lax.fori_loop(0, CHUNK, perm_body, 0, unroll=4)

        pltpu.sync_copy(buf_a, out_hbm.at[row])


@jax.jit
def kernel(x):
    mesh = plsc.VectorSubcoreMesh(
        core_axis_name="c", subcore_axis_name="s", num_cores=NC, num_subcores=NS
    )
    run = pl.kernel(
        _sort_body,
        out_type=jax.ShapeDtypeStruct((ROWS, N), jnp.float32),
        mesh=mesh,
        scratch_types=[
            pltpu.VMEM((N,), jnp.float32),
            pltpu.VMEM((N,), jnp.float32),
            pltpu.VMEM((HIST,), jnp.int32),
        ],
        compiler_params=pltpu.CompilerParams(needs_layout_passes=False),
    )
    return run(x)
